# baseline (speedup 1.0000x reference)
import functools

import jax
import jax.numpy as jnp
from jax import lax
from jax.experimental import pallas as pl
from jax.experimental.pallas import tpu as pltpu
from jax.experimental.pallas import tpu_sc as plsc

N_NODES = 1000000
EMBED_DIM = 64
BATCH = 16384
SLAB = 2 * EMBED_DIM

NUM_CORES = 2
NUM_SUBCORES = 16
NUM_TILES = NUM_CORES * NUM_SUBCORES
ROWS_PER_TILE = BATCH // NUM_TILES
CHUNK = 256

REPACK_ROWS = 10000


def _repack_kernel(top_ref, bot_ref, o_ref):
  o_ref[:, :EMBED_DIM] = top_ref[...]
  o_ref[:, EMBED_DIM:] = bot_ref[...]


def _tc_repack(weight):
  half = N_NODES // 2
  nblk = half // REPACK_ROWS
  return pl.pallas_call(
      _repack_kernel,
      grid=(nblk,),
      in_specs=[
          pl.BlockSpec((REPACK_ROWS, EMBED_DIM), lambda i: (i, 0)),
          pl.BlockSpec((REPACK_ROWS, EMBED_DIM), lambda i: (i + nblk, 0)),
      ],
      out_specs=pl.BlockSpec((REPACK_ROWS, SLAB), lambda i: (i, 0)),
      out_shape=jax.ShapeDtypeStruct((half, SLAB), jnp.float32),
  )(weight, weight)


def _sc_gather(w2, rx2, tx2):
  mesh = plsc.VectorSubcoreMesh(core_axis_name="c", subcore_axis_name="s")
  out_sds = jax.ShapeDtypeStruct((BATCH, SLAB), jnp.float32)

  @functools.partial(
      pl.kernel,
      mesh=mesh,
      out_type=(out_sds, out_sds),
      scratch_types=[
          pltpu.VMEM((ROWS_PER_TILE,), jnp.int32),
          pltpu.VMEM((ROWS_PER_TILE,), jnp.int32),
          pltpu.VMEM((CHUNK, SLAB), jnp.float32),
          pltpu.VMEM((CHUNK, SLAB), jnp.float32),
          pltpu.SemaphoreType.DMA,
          pltpu.SemaphoreType.DMA,
      ],
  )
  def k(w_hbm, rx_hbm, tx_hbm, a_hbm, b_hbm, rxi_v, txi_v, a_v, b_v, sa, sb):
    wid = lax.axis_index("s") * NUM_CORES + lax.axis_index("c")
    base = wid * ROWS_PER_TILE
    pltpu.sync_copy(rx_hbm.at[pl.ds(base, ROWS_PER_TILE)], rxi_v)
    pltpu.sync_copy(tx_hbm.at[pl.ds(base, ROWS_PER_TILE)], txi_v)

    @pl.loop(0, ROWS_PER_TILE, step=CHUNK)
    def _(r0):
      cp_a = pltpu.async_copy(w_hbm.at[rxi_v.at[pl.ds(r0, CHUNK)]], a_v, sa)
      cp_b = pltpu.async_copy(w_hbm.at[txi_v.at[pl.ds(r0, CHUNK)]], b_v, sb)
      cp_a.wait()
      cp_b.wait()
      pltpu.sync_copy(a_v, a_hbm.at[pl.ds(base + r0, CHUNK)])
      pltpu.sync_copy(b_v, b_hbm.at[pl.ds(base + r0, CHUNK)])

  return k(w2, rx2, tx2)


def _tc_kernel(a_ref, b_ref, pa_ref, pb_ref, bias_ref, o_ref):
  a = a_ref[...]
  b = b_ref[...]
  e_rx = jnp.where(pa_ref[...][:, None] == 1, a[:, EMBED_DIM:], a[:, :EMBED_DIM])
  e_tx = jnp.where(pb_ref[...][:, None] == 1, b[:, EMBED_DIM:], b[:, :EMBED_DIM])
  logits = jnp.sum(e_rx * e_tx, axis=1) + bias_ref[0]
  o_ref[...] = jax.nn.sigmoid(logits)


def _tc_combine(a, b, pa, pb, bias):
  block = 2048
  return pl.pallas_call(
      _tc_kernel,
      grid=(BATCH // block,),
      in_specs=[
          pl.BlockSpec((block, SLAB), lambda i: (i, 0)),
          pl.BlockSpec((block, SLAB), lambda i: (i, 0)),
          pl.BlockSpec((block,), lambda i: (i,)),
          pl.BlockSpec((block,), lambda i: (i,)),
          pl.BlockSpec((1,), lambda i: (0,)),
      ],
      out_specs=pl.BlockSpec((block,), lambda i: (i,)),
      out_shape=jax.ShapeDtypeStruct((BATCH,), jnp.float32),
  )(a, b, pa, pb, bias)


def kernel(rx, tx, weight, bias):
  rx = rx.astype(jnp.int32)
  tx = tx.astype(jnp.int32)
  half = N_NODES // 2
  pa = (rx >= half).astype(jnp.int32)
  pb = (tx >= half).astype(jnp.int32)
  w2 = _tc_repack(weight)
  a, b = _sc_gather(w2, rx - pa * half, tx - pb * half)
  return _tc_combine(a, b, pa, pb, bias)


# speedup vs baseline: 1.6578x; 1.6578x over previous
import functools

import jax
import jax.numpy as jnp
from jax import lax
from jax.experimental import pallas as pl
from jax.experimental.pallas import tpu as pltpu
from jax.experimental.pallas import tpu_sc as plsc

N_NODES = 1000000
EMBED_DIM = 64
BATCH = 16384

NUM_CORES = 2
NUM_SUBCORES = 16
NUM_LANES = 16
NUM_TILES = NUM_CORES * NUM_SUBCORES
ROWS_PER_TILE = BATCH // NUM_TILES
CHUNK = 256
UNROLL = 8


def _sc_gather_mul(weight, rx, tx):
  mesh = plsc.VectorSubcoreMesh(core_axis_name="c", subcore_axis_name="s")

  @functools.partial(
      pl.kernel,
      mesh=mesh,
      out_type=jax.ShapeDtypeStruct((BATCH, EMBED_DIM), jnp.float32),
      scratch_types=[
          pltpu.VMEM((ROWS_PER_TILE,), jnp.int32),
          pltpu.VMEM((ROWS_PER_TILE,), jnp.int32),
          pltpu.VMEM((CHUNK, EMBED_DIM), jnp.float32),
          pltpu.VMEM((CHUNK, EMBED_DIM), jnp.float32),
          pltpu.SemaphoreType.DMA,
          pltpu.SemaphoreType.DMA,
          pltpu.SemaphoreType.DMA,
          pltpu.SemaphoreType.DMA,
          pltpu.SemaphoreType.DMA,
          pltpu.SemaphoreType.DMA,
          pltpu.SemaphoreType.DMA,
          pltpu.SemaphoreType.DMA,
      ],
  )
  def k(w_hbm, rx_hbm, tx_hbm, out_hbm, rxi_v, txi_v, a_v, b_v,
        sa0, sa1, sa2, sa3, sb0, sb1, sb2, sb3):
    sas = (sa0, sa1, sa2, sa3)
    sbs = (sb0, sb1, sb2, sb3)
    wid = lax.axis_index("s") * NUM_CORES + lax.axis_index("c")
    base = wid * ROWS_PER_TILE
    pltpu.sync_copy(rx_hbm.at[pl.ds(base, ROWS_PER_TILE)], rxi_v)
    pltpu.sync_copy(tx_hbm.at[pl.ds(base, ROWS_PER_TILE)], txi_v)

    @pl.loop(0, ROWS_PER_TILE, step=CHUNK)
    def _(r0):
      @pl.loop(0, CHUNK, step=NUM_LANES)
      def _(i0):
        rv = rxi_v.at[pl.ds(r0 + i0, NUM_LANES)][...]
        tv = txi_v.at[pl.ds(r0 + i0, NUM_LANES)][...]
        for j in range(NUM_LANES):
          pltpu.async_copy(w_hbm.at[rv[j]], a_v.at[i0 + j], sas[j % 4])
          pltpu.async_copy(w_hbm.at[tv[j]], b_v.at[i0 + j], sbs[j % 4])

      @pl.loop(0, CHUNK, step=UNROLL)
      def _(i0):
        for j in range(UNROLL):
          i = i0 + j
          pltpu.make_async_copy(w_hbm.at[0], a_v.at[i], sas[j % 4]).wait()
          pltpu.make_async_copy(w_hbm.at[0], b_v.at[i], sbs[j % 4]).wait()

      @pl.loop(0, CHUNK)
      def _(i):
        for c in range(EMBED_DIM // NUM_LANES):
          slc = pl.ds(c * NUM_LANES, NUM_LANES)
          a_v.at[i, slc][...] = a_v.at[i, slc][...] * b_v.at[i, slc][...]

      pltpu.sync_copy(a_v, out_hbm.at[pl.ds(base + r0, CHUNK)])

  return k(weight, rx, tx)


def _tc_kernel(p_ref, b_ref, o_ref):
  o_ref[...] = jax.nn.sigmoid(jnp.sum(p_ref[...], axis=1) + b_ref[0])


def _tc_reduce_sigmoid(prod, bias):
  block = 2048
  return pl.pallas_call(
      _tc_kernel,
      grid=(BATCH // block,),
      in_specs=[
          pl.BlockSpec((block, EMBED_DIM), lambda i: (i, 0)),
          pl.BlockSpec((1,), lambda i: (0,)),
      ],
      out_specs=pl.BlockSpec((block,), lambda i: (i,)),
      out_shape=jax.ShapeDtypeStruct((BATCH,), jnp.float32),
  )(prod, bias)


def kernel(rx, tx, weight, bias):
  rx = rx.astype(jnp.int32)
  tx = tx.astype(jnp.int32)
  prod = _sc_gather_mul(weight, rx, tx)
  return _tc_reduce_sigmoid(prod, bias)


# speedup vs baseline: 1.6781x; 1.0122x over previous
import functools

import jax
import jax.numpy as jnp
from jax import lax
from jax.experimental import pallas as pl
from jax.experimental.pallas import tpu as pltpu
from jax.experimental.pallas import tpu_sc as plsc

N_NODES = 1000000
EMBED_DIM = 64
BATCH = 16384

NUM_CORES = 2
NUM_SUBCORES = 16
NUM_LANES = 16
NUM_TILES = NUM_CORES * NUM_SUBCORES
ROWS_PER_TILE = BATCH // NUM_TILES
CHUNK = 256
UNROLL = 8


def _sc_gather_mul(weight, rx, tx):
  mesh = plsc.VectorSubcoreMesh(core_axis_name="c", subcore_axis_name="s")

  @functools.partial(
      pl.kernel,
      mesh=mesh,
      out_type=jax.ShapeDtypeStruct((BATCH, EMBED_DIM), jnp.float32),
      scratch_types=[
          pltpu.VMEM((ROWS_PER_TILE,), jnp.int32),
          pltpu.VMEM((ROWS_PER_TILE,), jnp.int32),
          pltpu.VMEM((CHUNK, EMBED_DIM), jnp.float32),
          pltpu.VMEM((CHUNK, EMBED_DIM), jnp.float32),
          pltpu.SemaphoreType.DMA,
          pltpu.SemaphoreType.DMA,
          pltpu.SemaphoreType.DMA,
          pltpu.SemaphoreType.DMA,
          pltpu.SemaphoreType.DMA,
          pltpu.SemaphoreType.DMA,
          pltpu.SemaphoreType.DMA,
          pltpu.SemaphoreType.DMA,
      ],
  )
  def k(w_hbm, rx_hbm, tx_hbm, out_hbm, rxi_v, txi_v, a_v, b_v,
        sa0, sa1, sa2, sa3, sb0, sb1, sb2, sb3):
    sas = (sa0, sa1, sa2, sa3)
    sbs = (sb0, sb1, sb2, sb3)
    wid = lax.axis_index("s") * NUM_CORES + lax.axis_index("c")
    base = wid * ROWS_PER_TILE
    pltpu.sync_copy(rx_hbm.at[pl.ds(base, ROWS_PER_TILE)], rxi_v)
    pltpu.sync_copy(tx_hbm.at[pl.ds(base, ROWS_PER_TILE)], txi_v)

    @pl.loop(0, ROWS_PER_TILE, step=CHUNK)
    def _(r0):
      @pl.loop(0, CHUNK, step=NUM_LANES)
      def _(i0):
        rv = rxi_v.at[pl.ds(r0 + i0, NUM_LANES)][...]
        tv = txi_v.at[pl.ds(r0 + i0, NUM_LANES)][...]
        for j in range(NUM_LANES):
          pltpu.async_copy(w_hbm.at[rv[j]], a_v.at[i0 + j], sas[j % 4])
          pltpu.async_copy(w_hbm.at[tv[j]], b_v.at[i0 + j], sbs[j % 4])

      @pl.loop(0, CHUNK, step=UNROLL)
      def _(i0):
        for j in range(UNROLL):
          i = i0 + j
          pltpu.make_async_copy(w_hbm.at[0], a_v.at[i], sas[j % 4]).wait()
          pltpu.make_async_copy(w_hbm.at[0], b_v.at[i], sbs[j % 4]).wait()

      @pl.loop(0, CHUNK)
      def _(i):
        for c in range(EMBED_DIM // NUM_LANES):
          slc = pl.ds(c * NUM_LANES, NUM_LANES)
          a_v.at[i, slc][...] = a_v.at[i, slc][...] * b_v.at[i, slc][...]

      pltpu.sync_copy(a_v, out_hbm.at[pl.ds(base + r0, CHUNK)])

  return k(weight, rx, tx)


def _tc_kernel(p_ref, b_ref, o_ref):
  o_ref[...] = jax.nn.sigmoid(jnp.sum(p_ref[...], axis=1) + b_ref[0])


def _tc_reduce_sigmoid(prod, bias):
  block = 2048
  return pl.pallas_call(
      _tc_kernel,
      grid=(BATCH // block,),
      in_specs=[
          pl.BlockSpec((block, EMBED_DIM), lambda i: (i, 0)),
          pl.BlockSpec((1,), lambda i: (0,)),
      ],
      out_specs=pl.BlockSpec((block,), lambda i: (i,)),
      out_shape=jax.ShapeDtypeStruct((BATCH,), jnp.float32),
  )(prod, bias)


def kernel(rx, tx, weight, bias):
  rx = rx.astype(jnp.int32)
  tx = tx.astype(jnp.int32)
  prod = _sc_gather_mul(weight, rx, tx)
  return jax.nn.sigmoid(jnp.sum(prod, axis=1) + bias)
